# baseline (speedup 1.0000x reference)
import functools

import jax
import jax.numpy as jnp
from jax.experimental import pallas as pl
from jax.experimental.pallas import tpu as pltpu


def _vq_block(z_blk, cb, q_ref, cnt_ref):
    z2 = jnp.sum(z_blk * z_blk, axis=0)
    cb2 = jnp.sum(cb * cb, axis=1)
    scores = jax.lax.dot_general(
        -2.0 * cb, z_blk,
        dimension_numbers=(((1,), (0,)), ((), ())),
        preferred_element_type=jnp.float32,
    )
    d = (z2[None, :] + scores) + cb2[:, None]

    idx = jnp.argmin(d, axis=0)
    p = (jax.lax.broadcasted_iota(jnp.int32, d.shape, 0)
         == idx[None, :]).astype(jnp.bfloat16)

    q_ref[...] = jax.lax.dot_general(
        cb.astype(jnp.bfloat16), p,
        dimension_numbers=(((0,), (0,)), ((), ())),
        preferred_element_type=jnp.float32,
    )

    f = p
    while f.shape[1] > 128:
        h = f.shape[1] // 2
        f = f[:, :h] + f[:, h:]
    cnt_ref[...] += f.astype(jnp.float32)


def _vq_body(z0_ref, z1_ref, cb_ref, q0_ref, q1_ref, perp_ref,
             cnt0_ref, cnt1_ref, *, nt0, n0, n1):
    b = pl.program_id(0)
    t = pl.program_id(1)
    nb = pl.num_programs(0)
    nt = pl.num_programs(1)

    @pl.when(jnp.logical_and(b == 0, t == 0))
    def _init():
        cnt0_ref[...] = jnp.zeros_like(cnt0_ref)
        cnt1_ref[...] = jnp.zeros_like(cnt1_ref)

    @pl.when(t < nt0)
    def _pair0():
        for h in range(z0_ref.shape[0]):
            _vq_block(z0_ref[h], cb_ref[0], q0_ref.at[h], cnt0_ref)

    @pl.when(t >= nt0)
    def _pair1():
        for h in range(z1_ref.shape[0]):
            _vq_block(z1_ref[h], cb_ref[1], q1_ref.at[h], cnt1_ref)

    @pl.when(jnp.logical_and(b == nb - 1, t == nt - 1))
    def _finalize():
        pm0 = jnp.sum(cnt0_ref[...], axis=1) / float(n0)
        pm1 = jnp.sum(cnt1_ref[...], axis=1) / float(n1)
        p0 = jnp.exp(-jnp.sum(pm0 * jnp.log(pm0 + 1e-10)))
        p1 = jnp.exp(-jnp.sum(pm1 * jnp.log(pm1 + 1e-10)))
        perp_ref[...] = jnp.concatenate(
            [p0.reshape(1, 1), p1.reshape(1, 1)], axis=1)


def kernel(z0, z1, codebook0, codebook1):
    B, C, T0 = z0.shape
    T1 = z1.shape[2]
    K = codebook0.shape[0]
    tt = 2048
    nt0, nt1 = T0 // tt, T1 // tt
    nt = nt0 + nt1
    cbs = jnp.stack([codebook0, codebook1])
    body = functools.partial(_vq_body, nt0=nt0, n0=B * T0, n1=B * T1)
    q0, q1, perp = pl.pallas_call(
        body,
        grid=(B // 8, nt),
        in_specs=[
            pl.BlockSpec((8, C, tt),
                         lambda b, t: (b, 0, jnp.minimum(t, nt0 - 1))),
            pl.BlockSpec((8, C, tt),
                         lambda b, t: (b, 0, jnp.maximum(t - nt0, 0))),
            pl.BlockSpec((2, K, C), lambda b, t: (0, 0, 0)),
        ],
        out_specs=[
            pl.BlockSpec((8, C, tt),
                         lambda b, t: (b, 0, jnp.minimum(t, nt0 - 1))),
            pl.BlockSpec((8, C, tt),
                         lambda b, t: (b, 0, jnp.maximum(t - nt0, 0))),
            pl.BlockSpec((1, 2), lambda b, t: (0, 0)),
        ],
        out_shape=[
            jax.ShapeDtypeStruct((B, C, T0), jnp.float32),
            jax.ShapeDtypeStruct((B, C, T1), jnp.float32),
            jax.ShapeDtypeStruct((1, 2), jnp.float32),
        ],
        scratch_shapes=[
            pltpu.VMEM((K, 128), jnp.float32),
            pltpu.VMEM((K, 128), jnp.float32),
        ],
        compiler_params=pltpu.CompilerParams(
            dimension_semantics=("arbitrary", "arbitrary"),
        ),
    )(z0, z1, cbs)
    return (q0, q1, q0, q1, perp[0, 0], perp[0, 1])


# speedup vs baseline: 1.0050x; 1.0050x over previous
import functools

import jax
import jax.numpy as jnp
from jax.experimental import pallas as pl
from jax.experimental.pallas import tpu as pltpu


def _vq_block(z_blk, cb, q_ref, cnt_ref):
    z2 = jnp.sum(z_blk * z_blk, axis=0)
    cb2 = jnp.sum(cb * cb, axis=1)
    scores = jax.lax.dot_general(
        -2.0 * cb, z_blk,
        dimension_numbers=(((1,), (0,)), ((), ())),
        preferred_element_type=jnp.float32,
    )
    d = (z2[None, :] + scores) + cb2[:, None]

    idx = jnp.argmin(d, axis=0)
    p = (jax.lax.broadcasted_iota(jnp.int32, d.shape, 0)
         == idx[None, :]).astype(jnp.bfloat16)

    q_ref[...] = jax.lax.dot_general(
        cb.astype(jnp.bfloat16), p,
        dimension_numbers=(((0,), (0,)), ((), ())),
        preferred_element_type=jnp.float32,
    )

    f = p
    while f.shape[1] > 128:
        h = f.shape[1] // 2
        f = f[:, :h] + f[:, h:]
    cnt_ref[...] += f.astype(jnp.float32)


def _vq_body(z0_ref, z1_ref, cb_ref, q0_ref, q1_ref, perp_ref,
             cnt0_ref, cnt1_ref, *, nt0, n0, n1):
    b = pl.program_id(0)
    t = pl.program_id(1)
    nb = pl.num_programs(0)
    nt = pl.num_programs(1)

    @pl.when(jnp.logical_and(b == 0, t == 0))
    def _init():
        cnt0_ref[...] = jnp.zeros_like(cnt0_ref)
        cnt1_ref[...] = jnp.zeros_like(cnt1_ref)

    @pl.when(t < nt0)
    def _pair0():
        for h in range(z0_ref.shape[0]):
            _vq_block(z0_ref[h], cb_ref[0], q0_ref.at[h], cnt0_ref)

    @pl.when(t >= nt0)
    def _pair1():
        for h in range(z1_ref.shape[0]):
            _vq_block(z1_ref[h], cb_ref[1], q1_ref.at[h], cnt1_ref)

    @pl.when(jnp.logical_and(b == nb - 1, t == nt - 1))
    def _finalize():
        pm0 = jnp.sum(cnt0_ref[...], axis=1) / float(n0)
        pm1 = jnp.sum(cnt1_ref[...], axis=1) / float(n1)
        p0 = jnp.exp(-jnp.sum(pm0 * jnp.log(pm0 + 1e-10)))
        p1 = jnp.exp(-jnp.sum(pm1 * jnp.log(pm1 + 1e-10)))
        perp_ref[...] = jnp.concatenate(
            [p0.reshape(1, 1), p1.reshape(1, 1)], axis=1)


def kernel(z0, z1, codebook0, codebook1):
    B, C, T0 = z0.shape
    T1 = z1.shape[2]
    K = codebook0.shape[0]
    tt = 2048
    nt0, nt1 = T0 // tt, T1 // tt
    nt = nt0 + nt1
    cbs = jnp.stack([codebook0, codebook1])
    body = functools.partial(_vq_body, nt0=nt0, n0=B * T0, n1=B * T1)
    q0, q1, perp = pl.pallas_call(
        body,
        grid=(B // 4, nt),
        in_specs=[
            pl.BlockSpec((4, C, tt),
                         lambda b, t: (b, 0, jnp.minimum(t, nt0 - 1))),
            pl.BlockSpec((4, C, tt),
                         lambda b, t: (b, 0, jnp.maximum(t - nt0, 0))),
            pl.BlockSpec((2, K, C), lambda b, t: (0, 0, 0)),
        ],
        out_specs=[
            pl.BlockSpec((4, C, tt),
                         lambda b, t: (b, 0, jnp.minimum(t, nt0 - 1))),
            pl.BlockSpec((4, C, tt),
                         lambda b, t: (b, 0, jnp.maximum(t - nt0, 0))),
            pl.BlockSpec((1, 2), lambda b, t: (0, 0)),
        ],
        out_shape=[
            jax.ShapeDtypeStruct((B, C, T0), jnp.float32),
            jax.ShapeDtypeStruct((B, C, T1), jnp.float32),
            jax.ShapeDtypeStruct((1, 2), jnp.float32),
        ],
        scratch_shapes=[
            pltpu.VMEM((K, 128), jnp.float32),
            pltpu.VMEM((K, 128), jnp.float32),
        ],
        compiler_params=pltpu.CompilerParams(
            dimension_semantics=("arbitrary", "arbitrary"),
        ),
    )(z0, z1, cbs)
    return (q0, q1, q0, q1, perp[0, 0], perp[0, 1])
